# baseline (speedup 1.0000x reference)
import jax
import jax.numpy as jnp
from jax.experimental import pallas as pl

PATCH = 32
DIM = 256
N = 256
GRIDW = 16
INV_TEMP = 10.0


def _match_body(p0_ref, p1_ref, w_ref, out_ref):
    p0 = p0_ref[0]
    p1 = p1_ref[0]
    w = w_ref[...]

    f0 = jnp.dot(p0, w, preferred_element_type=jnp.float32)
    f0 = f0 / (jnp.sqrt(jnp.sum(f0 * f0, axis=1, keepdims=True)) + 1e-6)
    f1 = jnp.dot(p1, w, preferred_element_type=jnp.float32)
    f1 = f1 / (jnp.sqrt(jnp.sum(f1 * f1, axis=1, keepdims=True)) + 1e-6)

    sim = jax.lax.dot_general(
        f0, f1, (((1,), (1,)), ((), ())), preferred_element_type=jnp.float32
    ) * INV_TEMP

    rmax = jnp.max(sim, axis=1, keepdims=True)
    e_r = jnp.exp(sim - rmax)
    sm_r = e_r / jnp.sum(e_r, axis=1, keepdims=True)
    cmax = jnp.max(sim, axis=0, keepdims=True)
    e_c = jnp.exp(sim - cmax)
    sm_c = e_c / jnp.sum(e_c, axis=0, keepdims=True)
    scores = sm_r * sm_c

    iota_j = jax.lax.broadcasted_iota(jnp.int32, (N, N), 1)
    iota_i = jax.lax.broadcasted_iota(jnp.int32, (N, N), 0)

    row_max = jnp.max(scores, axis=1, keepdims=True)
    i2j = jnp.min(
        jnp.where(scores == row_max, iota_j, N), axis=1, keepdims=True
    )
    col_max = jnp.max(scores, axis=0, keepdims=True)
    j2i = jnp.min(
        jnp.where(scores == col_max, iota_i, N), axis=0, keepdims=True
    )

    hit = iota_j == i2j
    back = jnp.sum(jnp.where(hit, j2i, 0), axis=1, keepdims=True)

    row_ids = jax.lax.broadcasted_iota(jnp.int32, (N, 1), 0)
    mutual = back == row_ids
    conf = row_max
    valid = jnp.logical_and(mutual, conf > 1e-6)
    wt = conf * valid.astype(jnp.float32)

    half = jnp.float32(PATCH // 2)
    xl = (row_ids % GRIDW).astype(jnp.float32) * PATCH + half
    yl = (row_ids // GRIDW).astype(jnp.float32) * PATCH + half
    xr = (i2j % GRIDW).astype(jnp.float32) * PATCH + half
    yr = (i2j // GRIDW).astype(jnp.float32) * PATCH + half

    out_ref[0] = jnp.concatenate(
        [xl * wt, yl * wt, xr * wt, yr * wt], axis=1
    )


def kernel(image0, image1, W_proj):
    B, H, Wd = image0.shape
    h, w = H // PATCH, Wd // PATCH
    p0 = image0.reshape(B, h, PATCH, w, PATCH).transpose(0, 1, 3, 2, 4)
    p0 = p0.reshape(B, h * w, PATCH * PATCH)
    p1 = image1.reshape(B, h, PATCH, w, PATCH).transpose(0, 1, 3, 2, 4)
    p1 = p1.reshape(B, h * w, PATCH * PATCH)

    return pl.pallas_call(
        _match_body,
        grid=(B,),
        in_specs=[
            pl.BlockSpec((1, N, PATCH * PATCH), lambda b: (b, 0, 0)),
            pl.BlockSpec((1, N, PATCH * PATCH), lambda b: (b, 0, 0)),
            pl.BlockSpec((PATCH * PATCH, DIM), lambda b: (0, 0)),
        ],
        out_specs=pl.BlockSpec((1, N, 4), lambda b: (b, 0, 0)),
        out_shape=jax.ShapeDtypeStruct((B, N, 4), jnp.float32),
    )(p0, p1, W_proj)


# speedup vs baseline: 3.2611x; 3.2611x over previous
import jax
import jax.numpy as jnp
from jax.experimental import pallas as pl

PATCH = 32
DIM = 256
N = 256
GRIDW = 16
INV_TEMP = 10.0


def _patchify(img):
    p = img.reshape(GRIDW, PATCH, GRIDW, PATCH)
    p = p.transpose(0, 2, 1, 3)
    return p.reshape(N, PATCH * PATCH)


def _features(img, w):
    p = _patchify(img)
    f = jnp.dot(p, w, preferred_element_type=jnp.float32)
    return f / (jnp.sqrt(jnp.sum(f * f, axis=1, keepdims=True)) + 1e-6)


def _match_body(i0_ref, i1_ref, w_ref, out_ref):
    w = w_ref[...]
    f0 = _features(i0_ref[0], w)
    f1 = _features(i1_ref[0], w)

    sim = jax.lax.dot_general(
        f0, f1, (((1,), (1,)), ((), ())), preferred_element_type=jnp.float32
    ) * INV_TEMP

    rmax = jnp.max(sim, axis=1, keepdims=True)
    e_r = jnp.exp(sim - rmax)
    sm_r = e_r / jnp.sum(e_r, axis=1, keepdims=True)
    cmax = jnp.max(sim, axis=0, keepdims=True)
    e_c = jnp.exp(sim - cmax)
    sm_c = e_c / jnp.sum(e_c, axis=0, keepdims=True)
    scores = sm_r * sm_c

    iota_j = jax.lax.broadcasted_iota(jnp.int32, (N, N), 1)
    iota_i = jax.lax.broadcasted_iota(jnp.int32, (N, N), 0)

    row_max = jnp.max(scores, axis=1, keepdims=True)
    i2j = jnp.min(jnp.where(scores == row_max, iota_j, N), axis=1, keepdims=True)
    col_max = jnp.max(scores, axis=0, keepdims=True)
    j2i = jnp.min(jnp.where(scores == col_max, iota_i, N), axis=0, keepdims=True)

    hit = iota_j == i2j
    back = jnp.sum(jnp.where(hit, j2i, 0), axis=1, keepdims=True)

    row_ids = jax.lax.broadcasted_iota(jnp.int32, (N, 1), 0)
    mutual = back == row_ids
    conf = row_max
    valid = jnp.logical_and(mutual, conf > 1e-6)
    wt = conf * valid.astype(jnp.float32)

    half = jnp.float32(PATCH // 2)
    xl = (row_ids % GRIDW).astype(jnp.float32) * PATCH + half
    yl = (row_ids // GRIDW).astype(jnp.float32) * PATCH + half
    xr = (i2j % GRIDW).astype(jnp.float32) * PATCH + half
    yr = (i2j // GRIDW).astype(jnp.float32) * PATCH + half

    out_ref[0] = jnp.concatenate([xl * wt, yl * wt, xr * wt, yr * wt], axis=1)


def kernel(image0, image1, W_proj):
    B, H, Wd = image0.shape
    return pl.pallas_call(
        _match_body,
        grid=(B,),
        in_specs=[
            pl.BlockSpec((1, H, Wd), lambda b: (b, 0, 0)),
            pl.BlockSpec((1, H, Wd), lambda b: (b, 0, 0)),
            pl.BlockSpec((PATCH * PATCH, DIM), lambda b: (0, 0)),
        ],
        out_specs=pl.BlockSpec((1, N, 4), lambda b: (b, 0, 0)),
        out_shape=jax.ShapeDtypeStruct((B, N, 4), jnp.float32),
    )(image0, image1, W_proj)


# speedup vs baseline: 4.6178x; 1.4160x over previous
import jax
import jax.numpy as jnp
from jax.experimental import pallas as pl

PATCH = 32
DIM = 256
N = 256
GRIDW = 16
INV_TEMP = 10.0


def _patchify(img):
    p = img.reshape(GRIDW, PATCH, GRIDW, PATCH)
    p = p.transpose(0, 2, 1, 3)
    return p.reshape(N, PATCH * PATCH)


def _features(img, w):
    x4 = img.reshape(GRIDW, PATCH, GRIDW, PATCH)
    w3 = w.reshape(PATCH, PATCH, DIM)
    c = jax.lax.dot_general(
        x4, w3, (((3,), (1,)), ((1,), (0,))),
        preferred_element_type=jnp.float32,
    )
    f = jnp.sum(c, axis=0).reshape(N, DIM)
    return f / (jnp.sqrt(jnp.sum(f * f, axis=1, keepdims=True)) + 1e-6)


def _match_body(i0_ref, i1_ref, w_ref, out_ref):
    w = w_ref[...]
    f0 = _features(i0_ref[0], w)
    f1 = _features(i1_ref[0], w)

    sim = jax.lax.dot_general(
        f0, f1, (((1,), (1,)), ((), ())), preferred_element_type=jnp.float32
    ) * INV_TEMP

    rmax = jnp.max(sim, axis=1, keepdims=True)
    e_r = jnp.exp(sim - rmax)
    sm_r = e_r / jnp.sum(e_r, axis=1, keepdims=True)
    cmax = jnp.max(sim, axis=0, keepdims=True)
    e_c = jnp.exp(sim - cmax)
    sm_c = e_c / jnp.sum(e_c, axis=0, keepdims=True)
    scores = sm_r * sm_c

    iota_j = jax.lax.broadcasted_iota(jnp.int32, (N, N), 1)
    iota_i = jax.lax.broadcasted_iota(jnp.int32, (N, N), 0)

    row_max = jnp.max(scores, axis=1, keepdims=True)
    i2j = jnp.min(jnp.where(scores == row_max, iota_j, N), axis=1, keepdims=True)
    col_max = jnp.max(scores, axis=0, keepdims=True)
    j2i = jnp.min(jnp.where(scores == col_max, iota_i, N), axis=0, keepdims=True)

    hit = iota_j == i2j
    back = jnp.sum(jnp.where(hit, j2i, 0), axis=1, keepdims=True)

    row_ids = jax.lax.broadcasted_iota(jnp.int32, (N, 1), 0)
    mutual = back == row_ids
    conf = row_max
    valid = jnp.logical_and(mutual, conf > 1e-6)
    wt = conf * valid.astype(jnp.float32)

    half = jnp.float32(PATCH // 2)
    xl = (row_ids % GRIDW).astype(jnp.float32) * PATCH + half
    yl = (row_ids // GRIDW).astype(jnp.float32) * PATCH + half
    xr = (i2j % GRIDW).astype(jnp.float32) * PATCH + half
    yr = (i2j // GRIDW).astype(jnp.float32) * PATCH + half

    out_ref[0] = jnp.concatenate([xl * wt, yl * wt, xr * wt, yr * wt], axis=1)


def kernel(image0, image1, W_proj):
    B, H, Wd = image0.shape
    return pl.pallas_call(
        _match_body,
        grid=(B,),
        in_specs=[
            pl.BlockSpec((1, H, Wd), lambda b: (b, 0, 0)),
            pl.BlockSpec((1, H, Wd), lambda b: (b, 0, 0)),
            pl.BlockSpec((PATCH * PATCH, DIM), lambda b: (0, 0)),
        ],
        out_specs=pl.BlockSpec((1, N, 4), lambda b: (b, 0, 0)),
        out_shape=jax.ShapeDtypeStruct((B, N, 4), jnp.float32),
    )(image0, image1, W_proj)
